# baseline (speedup 1.0000x reference)
import jax
import jax.numpy as jnp
from jax.experimental import pallas as pl
from jax.experimental.pallas import tpu as pltpu

_N_CHUNKS = 8


def _dma_copy(w_ref, o_ref, *sems):
    rows = o_ref.shape[0]
    chunk = rows // _N_CHUNKS
    copies = [
        pltpu.make_async_copy(
            w_ref.at[pl.ds(i * chunk, chunk), :],
            o_ref.at[pl.ds(i * chunk, chunk), :],
            sems[i],
        )
        for i in range(_N_CHUNKS)
    ]
    for c in copies:
        c.start()
    for c in copies:
        c.wait()


def kernel(x, W):
    seq_len = x.shape[1]
    n_model = W.shape[1]
    out = pl.pallas_call(
        _dma_copy,
        in_specs=[pl.BlockSpec(memory_space=pl.ANY)],
        out_specs=pl.BlockSpec(memory_space=pl.ANY),
        out_shape=jax.ShapeDtypeStruct((seq_len, n_model), W.dtype),
        scratch_shapes=[pltpu.SemaphoreType.DMA] * _N_CHUNKS,
    )(W)
    return out


# speedup vs baseline: 41.7887x; 41.7887x over previous
import jax
import jax.numpy as jnp
from jax.experimental import pallas as pl

_BLK = 1024


def _copy_block(w_ref, o_ref):
    o_ref[...] = w_ref[...]


def kernel(x, W):
    seq_len = x.shape[1]
    n_model = W.shape[1]
    out = pl.pallas_call(
        _copy_block,
        grid=(seq_len // _BLK,),
        in_specs=[pl.BlockSpec((_BLK, n_model), lambda i: (i, 0))],
        out_specs=pl.BlockSpec((_BLK, n_model), lambda i: (i, 0)),
        out_shape=jax.ShapeDtypeStruct((seq_len, n_model), W.dtype),
    )(W)
    return out


# speedup vs baseline: 47.9136x; 1.1466x over previous
import jax
import jax.numpy as jnp
from jax.experimental import pallas as pl

_BLK = 2048


def _copy_block(w_ref, o_ref):
    o_ref[...] = w_ref[...]


def kernel(x, W):
    seq_len = x.shape[1]
    n_model = W.shape[1]
    out = pl.pallas_call(
        _copy_block,
        grid=(seq_len // _BLK,),
        in_specs=[pl.BlockSpec((_BLK, n_model), lambda i: (i, 0))],
        out_specs=pl.BlockSpec((_BLK, n_model), lambda i: (i, 0)),
        out_shape=jax.ShapeDtypeStruct((seq_len, n_model), W.dtype),
    )(W)
    return out
